# baseline (speedup 1.0000x reference)
import functools

import jax
import jax.numpy as jnp
from jax import lax
from jax.experimental import pallas as pl
from jax.experimental.pallas import tpu as pltpu
from jax.experimental.pallas import tpu_sc as plsc


def _build_kernel(F, V, D, B):
    info = plsc.get_sparse_core_info()
    NC, NS = info.num_cores, info.num_subcores
    NW = NC * NS
    rows_per_w = B // NW
    CB = 16
    nchunk = rows_per_w // CB
    NGV = CB // 16
    KD = D // 16
    FD = F * D
    inv_n = 1.0 / float(FD)

    mesh = plsc.VectorSubcoreMesh(core_axis_name="c", subcore_axis_name="s")

    @functools.partial(
        pl.kernel,
        out_type=jax.ShapeDtypeStruct((B, FD), jnp.float32),
        mesh=mesh,
        compiler_params=pltpu.CompilerParams(use_tc_tiling_on_sc=True),
        scratch_types=[
            pltpu.VMEM((F, rows_per_w), jnp.int32),
            pltpu.VMEM((CB * 8, D), jnp.float32),
            pltpu.VMEM((CB * 8, D), jnp.float32),
            pltpu.VMEM((CB * 8, D), jnp.float32),
            pltpu.VMEM((CB * 8, D), jnp.float32),
            pltpu.VMEM((CB, FD), jnp.float32),
            pltpu.VMEM((FD,), jnp.float32),
            pltpu.VMEM((FD,), jnp.float32),
            pltpu.SemaphoreType.DMA,
            pltpu.SemaphoreType.DMA,
            pltpu.SemaphoreType.DMA,
            pltpu.SemaphoreType.DMA,
        ],
    )
    def body(xt_hbm, tab_hbm, gamma_hbm, beta_hbm, out_hbm,
             idb_v, boxa_v, boxb_v, boxc_v, boxd_v, st_v, gam_v, bet_v,
             sema, semb, semc, semd):
        wid = lax.axis_index("s") * NC + lax.axis_index("c")
        pltpu.sync_copy(gamma_hbm, gam_v)
        pltpu.sync_copy(beta_hbm, bet_v)
        lane = lax.iota(jnp.int32, 16)
        perms = [lane ^ sh for sh in (8, 4, 2, 1)]
        gdn = lax.GatherDimensionNumbers(
            offset_dims=(), collapsed_slice_dims=(0,), start_index_map=(0,))

        def lane_total(v):
            for p in perms:
                v = v + lax.gather(
                    v, p[:, None], dimension_numbers=gdn, slice_sizes=(1,),
                    mode=lax.GatherScatterMode.PROMISE_IN_BOUNDS)
            return v

        pltpu.sync_copy(xt_hbm.at[:, pl.ds(wid * rows_per_w, rows_per_w)],
                        idb_v)

        def chunk_body(c, carry):
            base = (wid * nchunk + c) * CB

            def fire(f, box, sem):
                for g in range(NGV):
                    iv = idb_v[f, pl.ds(c * CB + g * 16, 16)]
                    iv = jnp.minimum(jnp.maximum(iv, 0), V - 1)
                    for k in range(16):
                        j = g * 16 + k
                        v0 = pl.multiple_of(
                            f * V + lax.bitwise_and(iv[k], jnp.int32(-8)), 8)
                        pltpu.async_copy(
                            tab_hbm.at[pl.ds(v0, 8), :],
                            box.at[pl.ds(j * 8, 8)], sem)

            def drain(box, sem):
                pltpu.make_async_copy(
                    tab_hbm.at[pl.ds(0, CB * 8), :], box, sem).wait()

            def compact(f, box):
                for g in range(NGV):
                    iv = idb_v[f, pl.ds(c * CB + g * 16, 16)]
                    iv = jnp.minimum(jnp.maximum(iv, 0), V - 1)
                    for k in range(16):
                        j = g * 16 + k
                        row = j * 8 + lax.bitwise_and(iv[k], jnp.int32(7))
                        for kk in range(KD):
                            st_v[j, pl.ds(f * D + kk * 16, 16)] = (
                                box[row, pl.ds(kk * 16, 16)])

            boxes = [(boxa_v, sema), (boxb_v, semb), (boxc_v, semc),
                     (boxd_v, semd)]
            fire(0, boxa_v, sema)
            fire(1, boxb_v, semb)

            def field_quad(i, carry2):
                f0 = 4 * i
                for u in range(4):
                    f = f0 + u
                    nbox, nsem = boxes[(u + 2) % 4]

                    @pl.when(f + 2 < F)
                    def _(f=f, nbox=nbox, nsem=nsem):
                        fire(f + 2, nbox, nsem)

                    box, sem = boxes[u % 4]
                    drain(box, sem)
                    compact(f, box)
                return carry2

            lax.fori_loop(0, F // 4, field_quad, 0)
            for f in (F - 2, F - 1):
                box, sem = boxes[f % 4]
                drain(box, sem)
                compact(f, box)

            def row_body(j, carry2):
                def stat_body(f, sq):
                    s, q = sq
                    for k in range(KD):
                        v = st_v[j, pl.ds(f * D + k * 16, 16)]
                        s = s + v
                        q = q + v * v
                    return (s, q)

                zeros = jnp.zeros((16,), jnp.float32)
                s, q = lax.fori_loop(0, F, stat_body, (zeros, zeros))
                mean = lane_total(s) * inv_n
                var = lane_total(q) * inv_n - mean * mean
                av = var + 1e-5
                ii = lax.bitcast_convert_type(av, jnp.int32)
                ii = 0x5F3759DF - lax.shift_right_arithmetic(ii, 1)
                y = lax.bitcast_convert_type(ii, jnp.float32)
                y = y * (1.5 - 0.5 * av * y * y)
                y = y * (1.5 - 0.5 * av * y * y)
                y = y * (1.5 - 0.5 * av * y * y)
                c1 = y
                c0 = -mean * y

                def norm_body(f, carry3):
                    for k in range(KD):
                        sl = pl.ds(f * D + k * 16, 16)
                        v = st_v[j, sl]
                        t = v * c1 + c0
                        st_v[j, sl] = t * gam_v[sl] + bet_v[sl]
                    return carry3

                lax.fori_loop(0, F, norm_body, 0)
                return carry2

            lax.fori_loop(0, CB, row_body, 0)
            pltpu.sync_copy(st_v, out_hbm.at[pl.ds(base, CB)])
            return carry

        lax.fori_loop(0, nchunk, chunk_body, 0)

    return body


def kernel(x_cat, tables, gamma, beta):
    B, F = x_cat.shape
    _, V, D = tables.shape
    tab2 = tables.reshape(F * V, D)
    return _build_kernel(F, V, D, B)(x_cat.T, tab2, gamma, beta)


# speedup vs baseline: 1.0414x; 1.0414x over previous
import functools

import jax
import jax.numpy as jnp
from jax import lax
from jax.experimental import pallas as pl
from jax.experimental.pallas import tpu as pltpu
from jax.experimental.pallas import tpu_sc as plsc


def _build_kernel(F, V, D, B):
    info = plsc.get_sparse_core_info()
    NC, NS = info.num_cores, info.num_subcores
    NW = NC * NS
    rows_per_w = B // NW
    CB = 16
    nchunk = rows_per_w // CB
    NGV = CB // 16
    KD = D // 16
    FD = F * D
    inv_n = 1.0 / float(FD)

    mesh = plsc.VectorSubcoreMesh(core_axis_name="c", subcore_axis_name="s")

    @functools.partial(
        pl.kernel,
        out_type=jax.ShapeDtypeStruct((B, FD), jnp.float32),
        mesh=mesh,
        compiler_params=pltpu.CompilerParams(use_tc_tiling_on_sc=True),
        scratch_types=[
            pltpu.VMEM((F, rows_per_w), jnp.int32),
            pltpu.VMEM((CB * 8, D), jnp.float32),
            pltpu.VMEM((CB * 8, D), jnp.float32),
            pltpu.VMEM((CB * 8, D), jnp.float32),
            pltpu.VMEM((CB * 8, D), jnp.float32),
            pltpu.VMEM((CB, FD), jnp.float32),
            pltpu.VMEM((FD,), jnp.float32),
            pltpu.VMEM((FD,), jnp.float32),
            pltpu.SemaphoreType.DMA,
            pltpu.SemaphoreType.DMA,
            pltpu.SemaphoreType.DMA,
            pltpu.SemaphoreType.DMA,
        ],
    )
    def body(xt_hbm, tab_hbm, gamma_hbm, beta_hbm, out_hbm,
             idb_v, boxa_v, boxb_v, boxc_v, boxd_v, st_v, gam_v, bet_v,
             sema, semb, semc, semd):
        wid = lax.axis_index("s") * NC + lax.axis_index("c")
        pltpu.sync_copy(gamma_hbm, gam_v)
        pltpu.sync_copy(beta_hbm, bet_v)
        lane = lax.iota(jnp.int32, 16)
        perms = [lane ^ sh for sh in (8, 4, 2, 1)]
        gdn = lax.GatherDimensionNumbers(
            offset_dims=(), collapsed_slice_dims=(0,), start_index_map=(0,))

        def lane_total(v):
            for p in perms:
                v = v + lax.gather(
                    v, p[:, None], dimension_numbers=gdn, slice_sizes=(1,),
                    mode=lax.GatherScatterMode.PROMISE_IN_BOUNDS)
            return v

        pltpu.sync_copy(xt_hbm.at[:, pl.ds(wid * rows_per_w, rows_per_w)],
                        idb_v)

        def clip_body(f, carry):
            for g in range(rows_per_w // 16):
                r = idb_v[f, pl.ds(g * 16, 16)]
                idb_v[f, pl.ds(g * 16, 16)] = (
                    jnp.minimum(jnp.maximum(r, 0), V - 1))
            return carry

        lax.fori_loop(0, F, clip_body, 0)

        def chunk_body(c, carry):
            base = (wid * nchunk + c) * CB

            def fire(f, box, sem):
                for g in range(NGV):
                    iv = idb_v[f, pl.ds(c * CB + g * 16, 16)]
                    for k in range(16):
                        j = g * 16 + k
                        v0 = pl.multiple_of(
                            f * V + lax.bitwise_and(iv[k], jnp.int32(-8)), 8)
                        pltpu.async_copy(
                            tab_hbm.at[pl.ds(v0, 8), :],
                            box.at[pl.ds(j * 8, 8)], sem)

            def drain(box, sem):
                pltpu.make_async_copy(
                    tab_hbm.at[pl.ds(0, CB * 8), :], box, sem).wait()

            def compact(f, box):
                for g in range(NGV):
                    iv = idb_v[f, pl.ds(c * CB + g * 16, 16)]
                    for k in range(16):
                        j = g * 16 + k
                        row = j * 8 + lax.bitwise_and(iv[k], jnp.int32(7))
                        for kk in range(KD):
                            st_v[j, pl.ds(f * D + kk * 16, 16)] = (
                                box[row, pl.ds(kk * 16, 16)])

            boxes = [(boxa_v, sema), (boxb_v, semb), (boxc_v, semc),
                     (boxd_v, semd)]
            fire(0, boxa_v, sema)
            fire(1, boxb_v, semb)

            def field_quad(i, carry2):
                f0 = 4 * i
                for u in range(4):
                    f = f0 + u
                    nbox, nsem = boxes[(u + 2) % 4]

                    @pl.when(f + 2 < F)
                    def _(f=f, nbox=nbox, nsem=nsem):
                        fire(f + 2, nbox, nsem)

                    box, sem = boxes[u % 4]
                    drain(box, sem)
                    compact(f, box)
                return carry2

            lax.fori_loop(0, F // 4, field_quad, 0)
            for f in (F - 2, F - 1):
                box, sem = boxes[f % 4]
                drain(box, sem)
                compact(f, box)

            def row_body(j, carry2):
                def stat_body(h, sq):
                    s, q = sq
                    for u in range(2):
                        f = 2 * h + u
                        for k in range(KD):
                            v = st_v[j, pl.ds(f * D + k * 16, 16)]
                            s = s + v
                            q = q + v * v
                    return (s, q)

                zeros = jnp.zeros((16,), jnp.float32)
                s, q = lax.fori_loop(0, F // 2, stat_body, (zeros, zeros))
                mean = lane_total(s) * inv_n
                var = lane_total(q) * inv_n - mean * mean
                av = var + 1e-5
                ii = lax.bitcast_convert_type(av, jnp.int32)
                ii = 0x5F3759DF - lax.shift_right_arithmetic(ii, 1)
                y = lax.bitcast_convert_type(ii, jnp.float32)
                y = y * (1.5 - 0.5 * av * y * y)
                y = y * (1.5 - 0.5 * av * y * y)
                y = y * (1.5 - 0.5 * av * y * y)
                c1 = y
                c0 = -mean * y

                def norm_body(h, carry3):
                    for u in range(2):
                        f = 2 * h + u
                        for k in range(KD):
                            sl = pl.ds(f * D + k * 16, 16)
                            v = st_v[j, sl]
                            t = v * c1 + c0
                            st_v[j, sl] = t * gam_v[sl] + bet_v[sl]
                    return carry3

                lax.fori_loop(0, F // 2, norm_body, 0)
                return carry2

            lax.fori_loop(0, CB, row_body, 0)
            pltpu.sync_copy(st_v, out_hbm.at[pl.ds(base, CB)])
            return carry

        lax.fori_loop(0, nchunk, chunk_body, 0)

    return body


def kernel(x_cat, tables, gamma, beta):
    B, F = x_cat.shape
    _, V, D = tables.shape
    tab2 = tables.reshape(F * V, D)
    return _build_kernel(F, V, D, B)(x_cat.T, tab2, gamma, beta)
